# baseline (speedup 1.0000x reference)
import functools
import math

import jax
import jax.numpy as jnp
from jax import lax
from jax.experimental import pallas as pl
from jax.experimental.pallas import tpu as pltpu
from jax.experimental.pallas import tpu_sc as plsc

EMB = 128
LANES = 16
NC = 2
NS = 16
NW = NC * NS
CHUNK = 64
NBUF = 10
AHEAD = 8
SCALE = math.sqrt(float(EMB))


def _make_emb(B: int):
    assert B % (NW * CHUNK) == 0
    bpw = B // NW
    nchunk = bpw // CHUNK
    nouter = nchunk // NBUF
    assert nchunk % NBUF == 0 and nouter >= 3
    mesh = plsc.VectorSubcoreMesh(core_axis_name="c", subcore_axis_name="s")

    @functools.partial(
        pl.kernel,
        mesh=mesh,
        compiler_params=pltpu.CompilerParams(use_tc_tiling_on_sc=True),
        out_type=jax.ShapeDtypeStruct((B, EMB), jnp.float32),
        scratch_types=[
            pltpu.VMEM((nchunk, CHUNK), jnp.int32),
            pltpu.VMEM((NBUF, CHUNK, EMB), jnp.float32),
            [pltpu.SemaphoreType.DMA] * NBUF,
            [pltpu.SemaphoreType.DMA] * NBUF,
        ],
    )
    def emb(idx_hbm, table_hbm, out_hbm, idx_v, rows_v, gsems, ssems):
        wid = lax.axis_index("s") * NC + lax.axis_index("c")
        base = wid * bpw
        pltpu.sync_copy(idx_hbm.at[wid], idx_v)

        def issue_gather(h, bh):
            pltpu.async_copy(table_hbm.at[idx_v.at[h]], rows_v.at[bh], gsems[bh])

        def wait_gather(b):
            pltpu.make_async_copy(
                table_hbm.at[pl.ds(0, CHUNK)], rows_v.at[b], gsems[b]).wait()

        def wait_store(b):
            pltpu.make_async_copy(
                table_hbm.at[pl.ds(0, CHUNK)], rows_v.at[b], ssems[b]).wait()

        def scale_buf(b):
            def row_body(i, c):
                for j in range(EMB // LANES):
                    sl = pl.ds(j * LANES, LANES)
                    rows_v[b, i, sl] = rows_v[b, i, sl] * SCALE
                return c
            lax.fori_loop(0, CHUNK, row_body, 0)

        def slot(g, b, h, wait_h_store, issue_h):
            wait_gather(b)
            scale_buf(b)
            pltpu.async_copy(
                rows_v.at[b], out_hbm.at[pl.ds(base + g * CHUNK, CHUNK)],
                ssems[b])
            if issue_h:
                bh = (b + AHEAD) % NBUF
                if wait_h_store:
                    wait_store(bh)
                issue_gather(h, bh)

        for g in range(AHEAD):
            issue_gather(g, g)

        for b in range(NBUF):
            g = b
            h = g + AHEAD
            slot(g, b, h, wait_h_store=(h >= NBUF), issue_h=True)

        def outer_body(o, carry):
            g0 = o * NBUF
            for b in range(NBUF):
                slot(g0 + b, b, g0 + b + AHEAD, wait_h_store=True, issue_h=True)
            return carry

        lax.fori_loop(1, nouter - 1, outer_body, 0)

        g0 = (nouter - 1) * NBUF
        for b in range(NBUF):
            g = g0 + b
            h = g + AHEAD
            slot(g, b, h, wait_h_store=True, issue_h=(h < nchunk))

        for b in range(NBUF):
            wait_store(b)

    return emb


def kernel(tokens, table):
    s0, s1 = tokens.shape
    b = s0 * s1
    idx = tokens.astype(jnp.int32).reshape(NW, b // (NW * CHUNK), CHUNK)
    out = _make_emb(b)(idx, table)
    return out.reshape(s0, s1, EMB)


# speedup vs baseline: 3.1062x; 3.1062x over previous
import functools
import math

import jax
import jax.numpy as jnp
from jax import lax
from jax.experimental import pallas as pl
from jax.experimental.pallas import tpu as pltpu
from jax.experimental.pallas import tpu_sc as plsc

EMB = 128
LANES = 16
NC = 2
NS = 16
NW = NC * NS
CHUNK = 64
NBUF = 10
AHEAD = 8
SCALE = math.sqrt(float(EMB))


def _make_emb(B: int):
    assert B % (NW * CHUNK) == 0
    bpw = B // NW
    nchunk = bpw // CHUNK
    nouter = nchunk // NBUF
    assert nchunk % NBUF == 0 and nouter >= 3
    mesh = plsc.VectorSubcoreMesh(core_axis_name="c", subcore_axis_name="s")

    @functools.partial(
        pl.kernel,
        mesh=mesh,
        compiler_params=pltpu.CompilerParams(use_tc_tiling_on_sc=True),
        out_type=jax.ShapeDtypeStruct((B, EMB), jnp.float32),
        scratch_types=[
            pltpu.VMEM((nchunk, CHUNK), jnp.int32),
            pltpu.VMEM((NBUF, CHUNK, EMB), jnp.float32),
            [pltpu.SemaphoreType.DMA] * NBUF,
            [pltpu.SemaphoreType.DMA] * NBUF,
        ],
    )
    def emb(idx_hbm, table_hbm, out_hbm, idx_v, rows_v, gsems, ssems):
        wid = lax.axis_index("s") * NC + lax.axis_index("c")
        base = wid * bpw
        pltpu.sync_copy(idx_hbm.at[wid], idx_v)

        def issue_gather(h, bh):
            pltpu.async_copy(table_hbm.at[idx_v.at[h]], rows_v.at[bh], gsems[bh])

        def wait_gather(b):
            pltpu.make_async_copy(
                table_hbm.at[pl.ds(0, CHUNK)], rows_v.at[b], gsems[b]).wait()

        def wait_store(b):
            pltpu.make_async_copy(
                table_hbm.at[pl.ds(0, CHUNK)], rows_v.at[b], ssems[b]).wait()

        def scale_buf(b):
            def row_body(i, c):
                for j in range(EMB // LANES):
                    sl = pl.ds(j * LANES, LANES)
                    rows_v[b, i, sl] = rows_v[b, i, sl] * SCALE
                return c
            lax.fori_loop(0, CHUNK, row_body, 0)

        def slot(g, b, h, wait_h_store, issue_h):
            wait_gather(b)
            scale_buf(b)
            pltpu.async_copy(
                rows_v.at[b], out_hbm.at[pl.ds(base + g * CHUNK, CHUNK)],
                ssems[b])
            if issue_h:
                bh = (b + AHEAD) % NBUF
                if wait_h_store:
                    wait_store(bh)
                issue_gather(h, bh)

        for g in range(AHEAD):
            issue_gather(g, g)

        for b in range(NBUF):
            g = b
            h = g + AHEAD
            slot(g, b, h, wait_h_store=(h >= NBUF), issue_h=True)

        def outer_body(o, carry):
            g0 = o * NBUF
            for b in range(NBUF):
                slot(g0 + b, b, g0 + b + AHEAD, wait_h_store=True, issue_h=True)
            return carry

        lax.fori_loop(1, nouter - 1, outer_body, 0)

        g0 = (nouter - 1) * NBUF
        for b in range(NBUF):
            g = g0 + b
            h = g + AHEAD
            slot(g, b, h, wait_h_store=True, issue_h=(h < nchunk))

        for b in range(NBUF):
            wait_store(b)

    return emb


def kernel(tokens, table):
    s0, s1 = tokens.shape
    b = s0 * s1
    idx = tokens.astype(jnp.int32).T.reshape(NW, b // (NW * CHUNK), CHUNK)
    out = _make_emb(b)(idx, table)
    return out.reshape(s1, s0, EMB).transpose(1, 0, 2)
